# baseline (speedup 1.0000x reference)
import functools

import jax
import jax.numpy as jnp
from jax import lax
from jax.experimental import pallas as pl
from jax.experimental.pallas import tpu as pltpu
from jax.experimental.pallas import tpu_sc as plsc

E = 8
K = 2
TM = 512
NW = 32

CH_G = 64
CH_C = 16


def _router_body(x_ref, gw_ref, gb_ref,
                 logits_ref, wts_ref, etop_ref, rank_ref, counts_ref,
                 run_ref):
    m = pl.program_id(0)

    @pl.when(m == 0)
    def _init():
        run_ref[...] = jnp.zeros_like(run_ref)

    xs = x_ref[...]
    logits = lax.dot_general(
        xs, gw_ref[...], (((1,), (1,)), ((), ())),
        preferred_element_type=jnp.float32) + gb_ref[...]
    logits_ref[...] = logits
    probs = jax.nn.softmax(logits, axis=-1)
    iota_e = lax.broadcasted_iota(jnp.int32, probs.shape, 1)
    a1 = jnp.argmax(probs, axis=-1, keepdims=True)
    m1 = jnp.max(probs, axis=-1, keepdims=True)
    probs2 = jnp.where(iota_e == a1, -jnp.inf, probs)
    a2 = jnp.argmax(probs2, axis=-1, keepdims=True)
    m2 = jnp.max(probs2, axis=-1, keepdims=True)
    denom = m1 + m2
    wts_ref[...] = jnp.concatenate([m1 / denom, m2 / denom], axis=1)
    etop_ref[...] = jnp.concatenate([a1, a2], axis=1).astype(jnp.int32)

    oh1 = (iota_e == a1).astype(jnp.float32)
    oh2 = (iota_e == a2).astype(jnp.float32)
    ohsum = oh1 + oh2
    tm = xs.shape[0]
    ir = lax.broadcasted_iota(jnp.int32, (tm, tm), 0)
    ic = lax.broadcasted_iota(jnp.int32, (tm, tm), 1)
    tril = (ir > ic).astype(jnp.float32)
    base = lax.dot_general(
        tril, ohsum, (((1,), (0,)), ((), ())),
        preferred_element_type=jnp.float32)
    tot = run_ref[...] + base
    r1 = jnp.sum(oh1 * tot, axis=1, keepdims=True)
    r2 = jnp.sum(oh2 * tot, axis=1, keepdims=True)
    rank_ref[...] = jnp.concatenate([r1, r2], axis=1).astype(jnp.int32)
    run_new = run_ref[...] + jnp.sum(ohsum, axis=0, keepdims=True)
    run_ref[...] = run_new
    counts_ref[...] = run_new.astype(jnp.int32)


def _run_router(hs, gate_W, gb2):
    t, h = hs.shape
    grid = (t // TM,)
    return pl.pallas_call(
        _router_body,
        grid=grid,
        in_specs=[
            pl.BlockSpec((TM, h), lambda m: (m, 0)),
            pl.BlockSpec((E, h), lambda m: (0, 0)),
            pl.BlockSpec((1, E), lambda m: (0, 0)),
        ],
        out_specs=[
            pl.BlockSpec((TM, E), lambda m: (m, 0)),
            pl.BlockSpec((TM, K), lambda m: (m, 0)),
            pl.BlockSpec((TM, K), lambda m: (m, 0)),
            pl.BlockSpec((TM, K), lambda m: (m, 0)),
            pl.BlockSpec((1, E), lambda m: (0, 0)),
        ],
        out_shape=[
            jax.ShapeDtypeStruct((t, E), jnp.float32),
            jax.ShapeDtypeStruct((t, K), jnp.float32),
            jax.ShapeDtypeStruct((t, K), jnp.int32),
            jax.ShapeDtypeStruct((t, K), jnp.int32),
            jax.ShapeDtypeStruct((1, E), jnp.int32),
        ],
        scratch_shapes=[pltpu.VMEM((1, E), jnp.float32)],
        compiler_params=pltpu.CompilerParams(
            dimension_semantics=("arbitrary",),
        ),
    )(hs, gate_W, gb2)


def _sc_gather_body(x_ref, rows_ref, xs_ref, idx_v, buf, sem):
    wid = lax.axis_index("s") * 2 + lax.axis_index("c")
    n_per_w = xs_ref.shape[0] // NW
    base = wid * n_per_w

    def body(i, carry):
        off = pl.multiple_of(base + i * CH_G, CH_G)
        pltpu.sync_copy(rows_ref.at[pl.ds(off, CH_G)], idx_v)
        pltpu.async_copy(x_ref.at[idx_v], buf, sem).wait()
        pltpu.sync_copy(buf, xs_ref.at[pl.ds(off, CH_G)])
        return carry

    lax.fori_loop(0, n_per_w // CH_G, body, 0)


def _run_sc_gather(x_bf32, src_rows, np_rows, h2):
    mesh = plsc.VectorSubcoreMesh(core_axis_name="c", subcore_axis_name="s")
    fn = functools.partial(
        pl.kernel, mesh=mesh,
        out_type=jax.ShapeDtypeStruct((np_rows, h2), jnp.int32),
        scratch_types=[
            pltpu.VMEM((CH_G,), jnp.int32),
            pltpu.VMEM((CH_G, h2), jnp.int32),
            pltpu.SemaphoreType.DMA,
        ],
        compiler_params=pltpu.CompilerParams(use_tc_tiling_on_sc=True),
    )(_sc_gather_body)
    return fn(x_bf32, src_rows)


def _gmm_body(te_ref, xs_ref, w_ref, b_ref, ws_ref, y_ref):
    del te_ref
    y = lax.dot_general(
        xs_ref[...], w_ref[0], (((1,), (1,)), ((), ())),
        preferred_element_type=jnp.float32) + b_ref[0]
    y_ref[...] = y * ws_ref[...]


def _run_gmm(tile_expert, xs, w_bf, b3, w_sorted, ntiles, h):
    np_rows = ntiles * TM
    grid_spec = pltpu.PrefetchScalarGridSpec(
        num_scalar_prefetch=1,
        grid=(ntiles,),
        in_specs=[
            pl.BlockSpec((TM, h), lambda i, te: (i, 0)),
            pl.BlockSpec((1, h, h), lambda i, te: (te[i], 0, 0)),
            pl.BlockSpec((1, 1, h), lambda i, te: (te[i], 0, 0)),
            pl.BlockSpec((TM, 1), lambda i, te: (i, 0)),
        ],
        out_specs=pl.BlockSpec((TM, h), lambda i, te: (i, 0)),
    )
    return pl.pallas_call(
        _gmm_body,
        grid_spec=grid_spec,
        out_shape=jax.ShapeDtypeStruct((np_rows, h), jnp.float32),
        compiler_params=pltpu.CompilerParams(
            dimension_semantics=("arbitrary",),
        ),
    )(tile_expert, xs, w_bf, b3, w_sorted)


def _sc_combine_body(y_ref, p0_ref, p1_ref, o_ref, i0, i1, b0, b1, sem):
    wid = lax.axis_index("s") * 2 + lax.axis_index("c")
    t_per_w = o_ref.shape[0] // NW
    base = wid * t_per_w

    def body(ci, carry):
        off = pl.multiple_of(base + ci * CH_C, CH_C)
        pltpu.sync_copy(p0_ref.at[pl.ds(off, CH_C)], i0)
        pltpu.sync_copy(p1_ref.at[pl.ds(off, CH_C)], i1)
        pltpu.async_copy(y_ref.at[i0], b0, sem).wait()
        pltpu.async_copy(y_ref.at[i1], b1, sem).wait()
        nvec = CH_C * b0.shape[1] // 16

        def vbody(j, c2):
            for u in range(8):
                jj = j * 8 + u
                r = jj // (b0.shape[1] // 16)
                col = (jj % (b0.shape[1] // 16)) * 16
                b0[r, pl.ds(col, 16)] = (
                    b0[r, pl.ds(col, 16)] + b1[r, pl.ds(col, 16)])
            return c2

        lax.fori_loop(0, nvec // 8, vbody, 0)
        pltpu.sync_copy(b0, o_ref.at[pl.ds(off, CH_C)])
        return carry

    lax.fori_loop(0, t_per_w // CH_C, body, 0)


def _run_sc_combine(y, p0, p1, t, h):
    mesh = plsc.VectorSubcoreMesh(core_axis_name="c", subcore_axis_name="s")
    fn = functools.partial(
        pl.kernel, mesh=mesh,
        out_type=jax.ShapeDtypeStruct((t, h), jnp.float32),
        scratch_types=[
            pltpu.VMEM((CH_C,), jnp.int32),
            pltpu.VMEM((CH_C,), jnp.int32),
            pltpu.VMEM((CH_C, h), jnp.float32),
            pltpu.VMEM((CH_C, h), jnp.float32),
            pltpu.SemaphoreType.DMA,
        ],
        compiler_params=pltpu.CompilerParams(use_tc_tiling_on_sc=True),
    )(_sc_combine_body)
    return fn(y, p0, p1)


@jax.jit
def kernel(x, gate_W, gate_b, W, b):
    bx, sx, h = x.shape
    t = bx * sx
    ntiles = t * K // TM + E
    np_rows = ntiles * TM

    hs = x.reshape(t, h)
    gb2 = gate_b.reshape(1, E)
    w_bf = W.astype(jnp.bfloat16)
    b3 = b.reshape(E, 1, h)
    x_bf = hs.astype(jnp.bfloat16)

    logits, wts2, etop, rank2, counts2 = _run_router(hs, gate_W, gb2)
    counts = counts2[0]

    tiles_e = (counts + TM - 1) // TM
    ptiles = jnp.cumsum(tiles_e)
    poff = (ptiles - tiles_e) * TM
    tid = jnp.arange(ntiles, dtype=jnp.int32)
    tile_expert = jnp.minimum(
        jnp.sum((tid[:, None] >= ptiles[None, :]).astype(jnp.int32), axis=1),
        E - 1).astype(jnp.int32)
    pos = poff[etop] + rank2
    flat_pos = pos.reshape(-1)
    tok2 = jnp.repeat(jnp.arange(t, dtype=jnp.int32), K)
    src_rows = jnp.zeros((np_rows,), jnp.int32).at[flat_pos].set(
        tok2, unique_indices=True)
    w_sorted = jnp.zeros((np_rows,), jnp.float32).at[flat_pos].set(
        wts2.reshape(-1), unique_indices=True)

    x_bf32 = lax.bitcast_convert_type(
        x_bf.reshape(t, h // 2, 2), jnp.int32)
    xs32 = _run_sc_gather(x_bf32, src_rows, np_rows, h // 2)
    xs = lax.bitcast_convert_type(xs32, jnp.bfloat16).reshape(np_rows, h)
    y = _run_gmm(tile_expert, xs, w_bf, b3, w_sorted[:, None], ntiles, h)
    out = _run_sc_combine(y, pos[:, 0], pos[:, 1], t, h)
    return out.reshape(bx, sx, h), logits


# speedup vs baseline: 2.6819x; 2.6819x over previous
import functools

import jax
import jax.numpy as jnp
from jax.experimental import pallas as pl
from jax.experimental.pallas import tpu as pltpu

E = 8
TM = 1024


def _moe_body(x_ref, gw_ref, gb_ref, w_ref, b_ref, out_ref, logits_ref,
              wts_ref):
    e = pl.program_id(1)

    @pl.when(e == 0)
    def _router():
        xs = x_ref[...]
        logits = jax.lax.dot_general(
            xs, gw_ref[...], (((1,), (1,)), ((), ())),
            preferred_element_type=jnp.float32) + gb_ref[...]
        logits_ref[...] = logits
        probs = jax.nn.softmax(logits, axis=-1)
        iota = jax.lax.broadcasted_iota(jnp.int32, probs.shape, 1)
        a1 = jnp.argmax(probs, axis=-1, keepdims=True)
        m1 = jnp.max(probs, axis=-1, keepdims=True)
        probs2 = jnp.where(iota == a1, -jnp.inf, probs)
        a2 = jnp.argmax(probs2, axis=-1, keepdims=True)
        m2 = jnp.max(probs2, axis=-1, keepdims=True)
        sel = (iota == a1) | (iota == a2)
        wts_ref[...] = jnp.where(sel, probs, 0.0) / (m1 + m2)

    xb = x_ref[...].astype(jnp.bfloat16)
    mm = jax.lax.dot_general(
        xb, w_ref[0], (((1,), (1,)), ((), ())),
        preferred_element_type=jnp.float32)
    iota = jax.lax.broadcasted_iota(jnp.int32, wts_ref.shape, 1)
    w_col = jnp.sum(wts_ref[...] * (iota == e), axis=1, keepdims=True)
    contrib = (mm + b_ref[0]) * w_col

    @pl.when(e == 0)
    def _init():
        out_ref[...] = contrib.astype(jnp.bfloat16)

    @pl.when(e != 0)
    def _acc():
        out_ref[...] += contrib.astype(jnp.bfloat16)


@jax.jit
def kernel(x, gate_W, gate_b, W, b):
    Bx, Sx, Hx = x.shape
    T = Bx * Sx
    hs = x.reshape(T, Hx)
    W_bf = W.astype(jnp.bfloat16)
    gb2 = gate_b.reshape(1, E)
    b3 = b.reshape(E, 1, Hx)

    grid = (T // TM, E)
    out, logits = pl.pallas_call(
        _moe_body,
        grid=grid,
        in_specs=[
            pl.BlockSpec((TM, Hx), lambda m, e: (m, 0)),
            pl.BlockSpec((E, Hx), lambda m, e: (0, 0)),
            pl.BlockSpec((1, E), lambda m, e: (0, 0)),
            pl.BlockSpec((1, Hx, Hx), lambda m, e: (e, 0, 0)),
            pl.BlockSpec((1, 1, Hx), lambda m, e: (e, 0, 0)),
        ],
        out_specs=[
            pl.BlockSpec((TM, Hx), lambda m, e: (m, 0)),
            pl.BlockSpec((TM, E), lambda m, e: (m, 0)),
        ],
        out_shape=[
            jax.ShapeDtypeStruct((T, Hx), jnp.bfloat16),
            jax.ShapeDtypeStruct((T, E), jnp.float32),
        ],
        scratch_shapes=[pltpu.VMEM((TM, E), jnp.float32)],
        compiler_params=pltpu.CompilerParams(
            dimension_semantics=("parallel", "arbitrary"),
        ),
    )(hs, gate_W, gb2, W_bf, b3)
    return out.astype(jnp.float32).reshape(Bx, Sx, Hx), logits
